# baseline (speedup 1.0000x reference)
import functools

import jax
import jax.numpy as jnp
from jax import lax
from jax.experimental import pallas as pl
from jax.experimental.pallas import tpu as pltpu
from jax.experimental.pallas import tpu_sc as plsc

NUM_CORES = 2
NUM_SUBCORES = 16
NW = NUM_CORES * NUM_SUBCORES

B_TOTAL = 16384 * 26
IDX_W = 128
CHUNKS = B_TOTAL // (NW * IDX_W)
PER_W = CHUNKS * IDX_W
DIM = 64


def _gather_body(table_hbm, idx_hbm, out_hbm, idx_v, rows_v, sem):
    wid = lax.axis_index("s") * NUM_CORES + lax.axis_index("c")
    pltpu.sync_copy(idx_hbm.at[pl.ds(wid * CHUNKS, CHUNKS)], idx_v)
    base = wid * PER_W

    def step(j, carry):
        pltpu.async_copy(table_hbm.at[idx_v.at[j]], rows_v, sem).wait()
        pltpu.sync_copy(rows_v, out_hbm.at[pl.ds(base + j * IDX_W, IDX_W)])
        return carry

    lax.fori_loop(0, CHUNKS, step, 0)


@functools.partial(jax.jit, static_argnames=())
def _gather(x2d, table):
    mesh = plsc.VectorSubcoreMesh(core_axis_name="c", subcore_axis_name="s")
    k = pl.kernel(
        _gather_body,
        mesh=mesh,
        out_type=jax.ShapeDtypeStruct((B_TOTAL, DIM), jnp.float32),
        scratch_types=[
            pltpu.VMEM((CHUNKS, IDX_W), jnp.int32),
            pltpu.VMEM((IDX_W, DIM), jnp.float32),
            pltpu.SemaphoreType.DMA,
        ],
        compiler_params=pltpu.CompilerParams(use_tc_tiling_on_sc=False),
    )
    return k(table, x2d)


def kernel(x, table):
    x2d = x.reshape(NW * CHUNKS, IDX_W).astype(jnp.int32)
    out = _gather(x2d, table)
    return out.reshape(x.shape[0], x.shape[1], DIM)


# speedup vs baseline: 1.0757x; 1.0757x over previous
import functools

import jax
import jax.numpy as jnp
from jax import lax
from jax.experimental import pallas as pl
from jax.experimental.pallas import tpu as pltpu
from jax.experimental.pallas import tpu_sc as plsc

NUM_CORES = 2
NUM_SUBCORES = 16
NW = NUM_CORES * NUM_SUBCORES

B_TOTAL = 16384 * 26
IDX_W = 128
CHUNKS = B_TOTAL // (NW * IDX_W)
PER_W = CHUNKS * IDX_W
DIM = 64

K = 4
GROUP_ROWS = K * IDX_W
NG = CHUNKS // K


def _fire_group(table_hbm, idx_v, buf, sem, g):
    for r in range(K):
        pltpu.async_copy(
            table_hbm.at[idx_v.at[g * K + r]],
            buf.at[pl.ds(r * IDX_W, IDX_W)],
            sem,
        )


def _wait_group(table_hbm, idx_v, buf, sem, g):
    for r in range(K):
        pltpu.make_async_copy(
            table_hbm.at[idx_v.at[g * K + r]],
            buf.at[pl.ds(r * IDX_W, IDX_W)],
            sem,
        ).wait()


def _gather_body(table_hbm, idx_hbm, out_hbm, idx_v, buf0, buf1,
                 gsem0, gsem1, wsem0, wsem1):
    wid = lax.axis_index("s") * NUM_CORES + lax.axis_index("c")
    pltpu.sync_copy(idx_hbm.at[pl.ds(wid * CHUNKS, CHUNKS)], idx_v)
    base = wid * PER_W

    def write_group(buf, sem, g):
        pltpu.async_copy(buf, out_hbm.at[pl.ds(base + g * GROUP_ROWS,
                                               GROUP_ROWS)], sem)

    def wait_write(buf, sem, g):
        pltpu.make_async_copy(buf, out_hbm.at[pl.ds(base + g * GROUP_ROWS,
                                                    GROUP_ROWS)], sem).wait()

    _fire_group(table_hbm, idx_v, buf0, gsem0, 0)

    def step(t, carry):
        g = 2 * t
        @pl.when(t > 0)
        def _():
            wait_write(buf1, wsem1, g - 1)
        _fire_group(table_hbm, idx_v, buf1, gsem1, g + 1)
        _wait_group(table_hbm, idx_v, buf0, gsem0, g)
        write_group(buf0, wsem0, g)
        @pl.when(t < NG // 2 - 1)
        def _():
            wait_write(buf0, wsem0, g)
            _fire_group(table_hbm, idx_v, buf0, gsem0, g + 2)
        _wait_group(table_hbm, idx_v, buf1, gsem1, g + 1)
        write_group(buf1, wsem1, g + 1)
        return carry

    lax.fori_loop(0, NG // 2, step, 0)
    wait_write(buf0, wsem0, NG - 2)
    wait_write(buf1, wsem1, NG - 1)


@jax.jit
def _gather(x2d, table):
    mesh = plsc.VectorSubcoreMesh(core_axis_name="c", subcore_axis_name="s")
    k = pl.kernel(
        _gather_body,
        mesh=mesh,
        out_type=jax.ShapeDtypeStruct((B_TOTAL, DIM), jnp.float32),
        scratch_types=[
            pltpu.VMEM((CHUNKS, IDX_W), jnp.int32),
            pltpu.VMEM((GROUP_ROWS, DIM), jnp.float32),
            pltpu.VMEM((GROUP_ROWS, DIM), jnp.float32),
            pltpu.SemaphoreType.DMA,
            pltpu.SemaphoreType.DMA,
            pltpu.SemaphoreType.DMA,
            pltpu.SemaphoreType.DMA,
        ],
        compiler_params=pltpu.CompilerParams(use_tc_tiling_on_sc=False),
    )
    return k(table, x2d)


def kernel(x, table):
    x2d = x.reshape(NW * CHUNKS, IDX_W).astype(jnp.int32)
    out = _gather(x2d, table)
    return out.reshape(x.shape[0], x.shape[1], DIM)
